# Initial kernel scaffold; baseline (speedup 1.0000x reference)
import jax
import jax.numpy as jnp
from jax.experimental import pallas as pl


def kernel(segmentId, pathSegmentFeat, edge_index, emb_table, W1, b1, W2, b2, Wgat, a_src, a_dst, w_out, b_out):
    raise NotImplementedError("write your pallas kernel here")


# speedup vs baseline: 4.0659x; 4.0659x over previous
import jax
import jax.numpy as jnp
from jax.experimental import pallas as pl
from jax.experimental.pallas import tpu as pltpu

SEQ_MAX_LEN = 50
EDGE_NUM = 100000
N_PATH = 10000
EMB_DIM = 48
FEAT_DIM = 5
HIDDEN = 280
PATH_DIM = 2 * HIDDEN
HEADS = 2
OUT_FEATS = 1024
HEAD_DIM = OUT_FEATS // HEADS

_BP = 1000


def _path_kernel(emb_ref, feat_ref, mask_ref, w1a_ref, w1b_ref, b1_ref,
                 w2_ref, b2_ref, wgat_ref, a_ref, wh_ref, esd_ref,
                 acc_ref, len_ref):
    l = pl.program_id(1)

    @pl.when(l == 0)
    def _init():
        acc_ref[...] = jnp.zeros_like(acc_ref)
        len_ref[...] = jnp.zeros_like(len_ref)

    xe = emb_ref[0, :, :]
    xf = feat_ref[0, :, :]
    m = mask_ref[0, :, :]
    hl = jnp.maximum(
        jnp.dot(xe, w1a_ref[...], preferred_element_type=jnp.float32)
        + jnp.dot(xf, w1b_ref[...], preferred_element_type=jnp.float32)
        + b1_ref[...], 0.0)
    acc_ref[...] += hl * m
    len_ref[...] += m

    @pl.when(l == SEQ_MAX_LEN - 1)
    def _finish():
        pooled = acc_ref[...] / jnp.maximum(len_ref[...], 1.0)
        pe = jnp.dot(pooled, w2_ref[...],
                     preferred_element_type=jnp.float32) + b2_ref[...]
        wh = jnp.dot(pe, wgat_ref[...], preferred_element_type=jnp.float32)
        wh_ref[...] = wh
        esd_ref[...] = jnp.dot(wh, a_ref[...],
                               preferred_element_type=jnp.float32)


def _out_kernel(agg_ref, wout_ref, bout_ref, hout_ref, pred_ref):
    x = agg_ref[...]
    h = jnp.where(x > 0, x, jnp.exp(jnp.minimum(x, 0.0)) - 1.0)
    hout_ref[...] = h
    pred_ref[...] = jnp.dot(h, wout_ref[...],
                            preferred_element_type=jnp.float32) + bout_ref[...]


@jax.jit
def kernel(segmentId, pathSegmentFeat, edge_index, emb_table, W1, b1, W2, b2,
           Wgat, a_src, a_dst, w_out, b_out):
    ids = segmentId.astype(jnp.int32)
    ids_t = ids.T
    emb_t = jnp.take(emb_table, ids_t, axis=0)
    feat_t = jnp.transpose(pathSegmentFeat, (1, 0, 2))
    mask_t = (ids_t != 0).astype(jnp.float32)[:, :, None]

    zeros = jnp.zeros((HEAD_DIM,), jnp.float32)
    a_mat = jnp.stack([
        jnp.concatenate([a_src[0], zeros]),
        jnp.concatenate([zeros, a_src[1]]),
        jnp.concatenate([a_dst[0], zeros]),
        jnp.concatenate([zeros, a_dst[1]]),
    ], axis=1)

    grid = N_PATH // _BP
    full = lambda shape: pl.BlockSpec(shape, lambda i, l: (0,) * len(shape))
    wh, esd = pl.pallas_call(
        _path_kernel,
        grid=(grid, SEQ_MAX_LEN),
        in_specs=[
            pl.BlockSpec((1, _BP, EMB_DIM), lambda i, l: (l, i, 0)),
            pl.BlockSpec((1, _BP, FEAT_DIM), lambda i, l: (l, i, 0)),
            pl.BlockSpec((1, _BP, 1), lambda i, l: (l, i, 0)),
            full((EMB_DIM, HIDDEN)),
            full((FEAT_DIM, HIDDEN)),
            full((1, HIDDEN)),
            full((HIDDEN, PATH_DIM)),
            full((1, PATH_DIM)),
            full((PATH_DIM, OUT_FEATS)),
            full((OUT_FEATS, 4)),
        ],
        out_specs=[
            pl.BlockSpec((_BP, OUT_FEATS), lambda i, l: (i, 0)),
            pl.BlockSpec((_BP, 4), lambda i, l: (i, 0)),
        ],
        out_shape=[
            jax.ShapeDtypeStruct((N_PATH, OUT_FEATS), jnp.float32),
            jax.ShapeDtypeStruct((N_PATH, 4), jnp.float32),
        ],
        scratch_shapes=[
            pltpu.VMEM((_BP, HIDDEN), jnp.float32),
            pltpu.VMEM((_BP, 1), jnp.float32),
        ],
    )(emb_t, feat_t, mask_t,
      W1[:EMB_DIM], W1[EMB_DIM:], b1[None, :], W2, b2[None, :], Wgat, a_mat)

    es = esd[:, 0:2]
    ed = esd[:, 2:4]
    src = edge_index[0]
    dst = edge_index[1]
    e = es[src] + ed[dst]
    e = jnp.where(e >= 0, e, 0.2 * e)
    emax = jax.ops.segment_max(e, dst, num_segments=N_PATH)
    emax = jnp.where(jnp.isfinite(emax), emax, 0.0)
    ee = jnp.exp(e - emax[dst])
    denom = jax.ops.segment_sum(ee, dst, num_segments=N_PATH)
    attn = ee / jnp.maximum(denom[dst], 1e-9)
    attn_full = jnp.repeat(attn, HEAD_DIM, axis=1)
    agg = jax.ops.segment_sum(wh[src] * attn_full, dst, num_segments=N_PATH)

    h_out, pred = pl.pallas_call(
        _out_kernel,
        grid=(grid,),
        in_specs=[
            pl.BlockSpec((_BP, OUT_FEATS), lambda i: (i, 0)),
            pl.BlockSpec((OUT_FEATS, 1), lambda i: (0, 0)),
            pl.BlockSpec((1, 1), lambda i: (0, 0)),
        ],
        out_specs=[
            pl.BlockSpec((_BP, OUT_FEATS), lambda i: (i, 0)),
            pl.BlockSpec((_BP, 1), lambda i: (i, 0)),
        ],
        out_shape=[
            jax.ShapeDtypeStruct((N_PATH, OUT_FEATS), jnp.float32),
            jax.ShapeDtypeStruct((N_PATH, 1), jnp.float32),
        ],
    )(agg, w_out, b_out[None, :])

    return (pred, h_out)
